# baseline (speedup 1.0000x reference)
import functools

import jax
import jax.numpy as jnp
from jax import lax
from jax.experimental import pallas as pl
from jax.experimental.pallas import tpu as pltpu
from jax.experimental.pallas import tpu_sc as plsc

NU = 5000
NI = 5000
N = 10000
H = 128
E = 320000
G = 10000

NC = 2
NS = 16
NW = NC * NS

NP = 10240
EP = 327680
EW = EP // NW
BP = 12288
TRASH = 10000

ROWS_PER_TILE = NP // NS

_mesh = plsc.VectorSubcoreMesh(core_axis_name="c", subcore_axis_name="s")

_f32 = jnp.float32
_i32 = jnp.int32


def _wid():
    return lax.axis_index("s") * NC + lax.axis_index("c")


def _pass_a(user_ids, item_ids, user_table, item_table, dstp, batp,
            ones128, zeros1d):
    @functools.partial(
        pl.kernel,
        out_type=(
            jax.ShapeDtypeStruct((NP, H), _f32),
            jax.ShapeDtypeStruct((NC, NP), _f32),
            jax.ShapeDtypeStruct((NC, NP), _f32),
        ),
        mesh=_mesh,
        scratch_types=[
            pltpu.VMEM((40,), _i32),
            pltpu.VMEM((40, H), _f32),
            pltpu.VMEM((128,), _i32),
            pltpu.VMEM((128,), _f32),
            pltpu.VMEM((ROWS_PER_TILE,), _f32),
            pltpu.VMEM_SHARED((NP,), _f32),
            pltpu.VMEM_SHARED((NP,), _f32),
        ],
    )
    def k(uid_hbm, iid_hbm, utab_hbm, itab_hbm, dstp_hbm, batp_hbm,
          ones_hbm, z1d_hbm, x_hbm, deg_hbm, cnt_hbm,
          idx_v, rows_v, eidx_v, ones_v, z1d_v, deg_sh, cnt_sh):
        cid = lax.axis_index("c")
        sid = lax.axis_index("s")
        wid = _wid()

        pltpu.sync_copy(z1d_hbm, z1d_v)
        pltpu.sync_copy(z1d_v, deg_sh.at[pl.ds(sid * ROWS_PER_TILE, ROWS_PER_TILE)])
        pltpu.sync_copy(z1d_v, cnt_sh.at[pl.ds(sid * ROWS_PER_TILE, ROWS_PER_TILE)])
        pltpu.sync_copy(ones_hbm, ones_v)

        @pl.when(wid < 16)
        def _():
            @pl.loop(0, 8)
            def _(j):
                c = wid * 8 + j

                @pl.when(c < 125)
                def _():
                    pltpu.sync_copy(uid_hbm.at[pl.ds(c * 40, 40)], idx_v)
                    pltpu.sync_copy(utab_hbm.at[idx_v], rows_v)
                    pltpu.sync_copy(rows_v, x_hbm.at[pl.ds(c * 40, 40)])

        @pl.when(wid >= 16)
        def _():
            @pl.loop(0, 8)
            def _(j):
                c = (wid - 16) * 8 + j

                @pl.when(c < 125)
                def _():
                    pltpu.sync_copy(iid_hbm.at[pl.ds(c * 40, 40)], idx_v)
                    pltpu.sync_copy(itab_hbm.at[idx_v], rows_v)
                    pltpu.sync_copy(rows_v, x_hbm.at[pl.ds(NU + c * 40, 40)])

        plsc.subcore_barrier()

        @pl.loop(0, EW // 128)
        def _(j):
            pltpu.sync_copy(dstp_hbm.at[pl.ds(wid * EW + j * 128, 128)], eidx_v)
            pltpu.sync_copy(ones_v, deg_sh.at[eidx_v], add=True)

        @pl.loop(0, BP // NW // 128)
        def _(j):
            pltpu.sync_copy(batp_hbm.at[pl.ds(wid * (BP // NW) + j * 128, 128)], eidx_v)
            pltpu.sync_copy(ones_v, cnt_sh.at[eidx_v], add=True)

        plsc.subcore_barrier()

        @pl.when(sid == 0)
        def _():
            pltpu.sync_copy(deg_sh, deg_hbm.at[cid])
            pltpu.sync_copy(cnt_sh, cnt_hbm.at[cid])

    return k(user_ids, item_ids, user_table, item_table, dstp, batp,
             ones128, zeros1d)


NCHUNK = EW // 128
NBLK = 20
NBUF = 2
Q0 = 160
Q1 = 0


def _pass_c(edges2, y, zfull):
    @functools.partial(
        pl.kernel,
        out_type=jax.ShapeDtypeStruct((NC, NP, H), _f32),
        mesh=_mesh,
        scratch_types=[
            pltpu.VMEM((NBLK, 2, 128), _i32),
            pltpu.VMEM_SHARED((NP, H), _f32),
        ]
        + [pltpu.VMEM((128, H), _f32)] * NBUF
        + [pltpu.SemaphoreType.DMA] * (2 * NBUF),
    )
    def k(edges_hbm, y_hbm, zf_hbm, zp_hbm, idx_v, z_sh, *bufs_sems):
        rows = bufs_sems[:NBUF]
        gsem = bufs_sems[NBUF:2 * NBUF]
        ssem = bufs_sems[2 * NBUF:]
        cid = lax.axis_index("c")
        sid = lax.axis_index("s")

        @pl.loop(0, ROWS_PER_TILE // 64)
        def _(m):
            off = sid * ROWS_PER_TILE + m * 64
            pltpu.sync_copy(zf_hbm.at[pl.ds(off, 64)], z_sh.at[pl.ds(off, 64)])

        plsc.subcore_barrier()

        nblk = jnp.where(cid == 0, Q0 // NBLK, Q1 // NBLK)
        base = jnp.where(cid == 0, sid * Q0, NS * Q0 + sid * Q1)

        @pl.loop(0, nblk)
        def _(blk):
            pltpu.sync_copy(
                edges_hbm.at[pl.ds(base + blk * NBLK, NBLK)], idx_v)
            for b in range(NBUF):
                pltpu.async_copy(y_hbm.at[idx_v.at[b, 0]], rows[b], gsem[b])

            @pl.loop(0, NBLK, step=NBUF)
            def _(m):
                for b in range(NBUF):
                    pltpu.make_async_copy(y_hbm.at[idx_v.at[m + b, 0]],
                                          rows[b], gsem[b]).wait()
                    pltpu.async_copy(rows[b], z_sh.at[idx_v.at[m + b, 1]],
                                     ssem[b], add=True)
                for b in range(NBUF):
                    pltpu.make_async_copy(rows[b], z_sh.at[idx_v.at[m + b, 1]],
                                          ssem[b]).wait()

                    @pl.when(m + NBUF + b < NBLK)
                    def _():
                        pltpu.async_copy(y_hbm.at[idx_v.at[m + NBUF + b, 0]],
                                         rows[b], gsem[b])

        plsc.subcore_barrier()

        pltpu.sync_copy(z_sh.at[pl.ds(sid * ROWS_PER_TILE, ROWS_PER_TILE)],
                        zp_hbm.at[cid, pl.ds(sid * ROWS_PER_TILE, ROWS_PER_TILE)])

    return k(edges2, y, zfull)


def _pass_e(x3, batp, zfull):
    @functools.partial(
        pl.kernel,
        out_type=jax.ShapeDtypeStruct((NC, NP, H), _f32),
        mesh=_mesh,
        scratch_types=[
            pltpu.VMEM((64,), _i32),
            pltpu.VMEM((64, H), _f32),
            pltpu.VMEM_SHARED((NP, H), _f32),
        ],
    )
    def k(x3_hbm, batp_hbm, zf_hbm, sp_hbm, bidx_v, rows_v, s_sh):
        cid = lax.axis_index("c")
        sid = lax.axis_index("s")
        wid = _wid()

        @pl.loop(0, ROWS_PER_TILE // 64)
        def _(m):
            off = sid * ROWS_PER_TILE + m * 64
            pltpu.sync_copy(zf_hbm.at[pl.ds(off, 64)], s_sh.at[pl.ds(off, 64)])

        plsc.subcore_barrier()

        @pl.loop(0, NP // NW // 64)
        def _(j):
            off = wid * (NP // NW) + j * 64
            pltpu.sync_copy(x3_hbm.at[pl.ds(off, 64)], rows_v)
            pltpu.sync_copy(batp_hbm.at[pl.ds(off, 64)], bidx_v)
            pltpu.sync_copy(rows_v, s_sh.at[bidx_v], add=True)

        plsc.subcore_barrier()

        pltpu.sync_copy(s_sh.at[pl.ds(sid * ROWS_PER_TILE, ROWS_PER_TILE)],
                        sp_hbm.at[cid, pl.ds(sid * ROWS_PER_TILE, ROWS_PER_TILE)])

    return k(x3, batp, zfull)


_BLK = 2048
_GRID = NP // _BLK

_row_spec = pl.BlockSpec((_BLK, H), lambda i: (i, 0))
_col_spec = pl.BlockSpec((_BLK, 1), lambda i: (i, 0))
_w_spec = pl.BlockSpec((H, H), lambda i: (0, 0))
_b_spec = pl.BlockSpec((1, H), lambda i: (0, 0))


def _b_body(x_ref, d0_ref, d1_ref, c0_ref, c1_ref,
            y_ref, dinv_ref, cntinv_ref):
    deg = d0_ref[...] + d1_ref[...] + 1.0
    dinv = lax.rsqrt(deg)
    dinv_ref[...] = dinv
    y_ref[...] = x_ref[...] * dinv
    cnt = c0_ref[...] + c1_ref[...]
    cntinv_ref[...] = 1.0 / jnp.maximum(cnt, 1.0)


def _pass_b(x, d0, d1, c0, c1):
    return pl.pallas_call(
        _b_body,
        grid=(_GRID,),
        in_specs=[_row_spec, _col_spec, _col_spec, _col_spec, _col_spec],
        out_specs=(_row_spec, _col_spec, _col_spec),
        out_shape=(
            jax.ShapeDtypeStruct((NP, H), _f32),
            jax.ShapeDtypeStruct((NP, 1), _f32),
            jax.ShapeDtypeStruct((NP, 1), _f32),
        ),
    )(x, d0, d1, c0, c1)


def _d_body(z0_ref, z1_ref, y_ref, dinv_ref, w_ref, b_ref, o_ref,
            *, relu, scale_out):
    dinv = dinv_ref[...]
    t = (z0_ref[...] + z1_ref[...] + y_ref[...]) * dinv
    m = jnp.dot(t, w_ref[...], preferred_element_type=_f32) + b_ref[...]
    if relu:
        m = jnp.maximum(m, 0.0)
    if scale_out:
        m = m * dinv
    o_ref[...] = m


def _pass_d(z0, z1, y, dinv, w, b, relu, scale_out):
    return pl.pallas_call(
        functools.partial(_d_body, relu=relu, scale_out=scale_out),
        grid=(_GRID,),
        in_specs=[_row_spec, _row_spec, _row_spec, _col_spec, _w_spec, _b_spec],
        out_specs=_row_spec,
        out_shape=jax.ShapeDtypeStruct((NP, H), _f32),
    )(z0, z1, y, dinv, w, b)


def _f_body(s0_ref, s1_ref, cntinv_ref, w_ref, b_ref, o_ref):
    t = (s0_ref[...] + s1_ref[...]) * cntinv_ref[...]
    o_ref[...] = jnp.dot(t, w_ref[...], preferred_element_type=_f32) + b_ref[...]


def _pass_f(s0, s1, cntinv, wl, bl):
    return pl.pallas_call(
        _f_body,
        grid=(_GRID,),
        in_specs=[_row_spec, _row_spec, _col_spec, _w_spec, _b_spec],
        out_specs=_row_spec,
        out_shape=jax.ShapeDtypeStruct((NP, H), _f32),
    )(s0, s1, cntinv, wl, bl)


def kernel(user_ids, item_ids, edge_index, batch, user_table, item_table,
           W1, b1, W2, b2, W3, b3, Wl, bl):
    srcp = jnp.concatenate([edge_index[0], jnp.zeros((EP - E,), _i32)])
    dstp = jnp.concatenate([edge_index[1], jnp.full((EP - E,), TRASH, _i32)])
    edges2 = jnp.stack([srcp.reshape(EP // 128, 128),
                        dstp.reshape(EP // 128, 128)], axis=1)
    batp = jnp.concatenate([batch, jnp.full((BP - N,), TRASH, _i32)])
    ones128 = jnp.ones((128,), _f32)
    zeros1d = jnp.zeros((ROWS_PER_TILE,), _f32)
    zfull = jnp.zeros((NP, H), _f32)

    x, degc, cntc = _pass_a(user_ids, item_ids, user_table, item_table,
                            dstp, batp, ones128, zeros1d)
    d0 = degc[0].reshape(NP, 1)
    d1 = degc[1].reshape(NP, 1)
    c0 = cntc[0].reshape(NP, 1)
    c1 = cntc[1].reshape(NP, 1)

    y, dinv, cntinv = _pass_b(x, d0, d1, c0, c1)

    for w, b, relu, scale_out in ((W1, b1, True, True),
                                  (W2, b2, True, True),
                                  (W3, b3, False, False)):
        zp = _pass_c(edges2, y, zfull)
        y = _pass_d(zp[0], zp[1], y, dinv, w, b.reshape(1, H),
                    relu, scale_out)

    sp = _pass_e(y, batp, zfull)
    out = _pass_f(sp[0], sp[1], cntinv, Wl, bl.reshape(1, H))
    return (out[:NU], out[NU:N])


# speedup vs baseline: 1.4124x; 1.4124x over previous
import functools

import jax
import jax.numpy as jnp
from jax import lax
from jax.experimental import pallas as pl
from jax.experimental.pallas import tpu as pltpu
from jax.experimental.pallas import tpu_sc as plsc

NU = 5000
NI = 5000
N = 10000
H = 128
E = 320000
G = 10000

NC = 2
NS = 16
NW = NC * NS

NP = 10240
EP = 327680
EW = EP // NW
BP = 12288
TRASH = 10000

ROWS_PER_TILE = NP // NS

_mesh = plsc.VectorSubcoreMesh(core_axis_name="c", subcore_axis_name="s")

_f32 = jnp.float32
_i32 = jnp.int32


def _wid():
    return lax.axis_index("s") * NC + lax.axis_index("c")


def _pass_a(user_ids, item_ids, user_table, item_table, dstp, batp,
            ones128, zeros1d):
    @functools.partial(
        pl.kernel,
        out_type=(
            jax.ShapeDtypeStruct((NP, H), _f32),
            jax.ShapeDtypeStruct((NC, NP), _f32),
            jax.ShapeDtypeStruct((NC, NP), _f32),
        ),
        mesh=_mesh,
        scratch_types=[
            pltpu.VMEM((40,), _i32),
            pltpu.VMEM((40, H), _f32),
            pltpu.VMEM((128,), _i32),
            pltpu.VMEM((128,), _f32),
            pltpu.VMEM((ROWS_PER_TILE,), _f32),
            pltpu.VMEM_SHARED((NP,), _f32),
            pltpu.VMEM_SHARED((NP,), _f32),
        ],
    )
    def k(uid_hbm, iid_hbm, utab_hbm, itab_hbm, dstp_hbm, batp_hbm,
          ones_hbm, z1d_hbm, x_hbm, deg_hbm, cnt_hbm,
          idx_v, rows_v, eidx_v, ones_v, z1d_v, deg_sh, cnt_sh):
        cid = lax.axis_index("c")
        sid = lax.axis_index("s")
        wid = _wid()

        pltpu.sync_copy(z1d_hbm, z1d_v)
        pltpu.sync_copy(z1d_v, deg_sh.at[pl.ds(sid * ROWS_PER_TILE, ROWS_PER_TILE)])
        pltpu.sync_copy(z1d_v, cnt_sh.at[pl.ds(sid * ROWS_PER_TILE, ROWS_PER_TILE)])
        pltpu.sync_copy(ones_hbm, ones_v)

        @pl.when(wid < 16)
        def _():
            @pl.loop(0, 8)
            def _(j):
                c = wid * 8 + j

                @pl.when(c < 125)
                def _():
                    pltpu.sync_copy(uid_hbm.at[pl.ds(c * 40, 40)], idx_v)
                    pltpu.sync_copy(utab_hbm.at[idx_v], rows_v)
                    pltpu.sync_copy(rows_v, x_hbm.at[pl.ds(c * 40, 40)])

        @pl.when(wid >= 16)
        def _():
            @pl.loop(0, 8)
            def _(j):
                c = (wid - 16) * 8 + j

                @pl.when(c < 125)
                def _():
                    pltpu.sync_copy(iid_hbm.at[pl.ds(c * 40, 40)], idx_v)
                    pltpu.sync_copy(itab_hbm.at[idx_v], rows_v)
                    pltpu.sync_copy(rows_v, x_hbm.at[pl.ds(NU + c * 40, 40)])

        plsc.subcore_barrier()

        @pl.loop(0, EW // 128)
        def _(j):
            pltpu.sync_copy(dstp_hbm.at[pl.ds(wid * EW + j * 128, 128)], eidx_v)
            pltpu.sync_copy(ones_v, deg_sh.at[eidx_v], add=True)

        @pl.loop(0, BP // NW // 128)
        def _(j):
            pltpu.sync_copy(batp_hbm.at[pl.ds(wid * (BP // NW) + j * 128, 128)], eidx_v)
            pltpu.sync_copy(ones_v, cnt_sh.at[eidx_v], add=True)

        plsc.subcore_barrier()

        @pl.when(sid == 0)
        def _():
            pltpu.sync_copy(deg_sh, deg_hbm.at[cid])
            pltpu.sync_copy(cnt_sh, cnt_hbm.at[cid])

    return k(user_ids, item_ids, user_table, item_table, dstp, batp,
             ones128, zeros1d)


NCHUNK = EW // 128
NBLK = 20
NBUF = 2
Q0 = 120
Q1 = 40


def _pass_c(edges2, y0, y1, zfull):
    @functools.partial(
        pl.kernel,
        out_type=jax.ShapeDtypeStruct((NC, NP, H), _f32),
        mesh=_mesh,
        scratch_types=[
            pltpu.VMEM((NBLK, 2, 128), _i32),
            pltpu.VMEM_SHARED((NP, H), _f32),
        ]
        + [pltpu.VMEM((128, H), _f32)] * NBUF
        + [pltpu.SemaphoreType.DMA] * (2 * NBUF),
    )
    def k(edges_hbm, y0_hbm, y1_hbm, zf_hbm, zp_hbm, idx_v, z_sh, *bufs_sems):
        rows = bufs_sems[:NBUF]
        gsem = bufs_sems[NBUF:2 * NBUF]
        ssem = bufs_sems[2 * NBUF:]
        cid = lax.axis_index("c")
        sid = lax.axis_index("s")

        @pl.loop(0, ROWS_PER_TILE // 64)
        def _(m):
            off = sid * ROWS_PER_TILE + m * 64
            pltpu.sync_copy(zf_hbm.at[pl.ds(off, 64)], z_sh.at[pl.ds(off, 64)])

        plsc.subcore_barrier()

        def run(y_hbm, nblk, base):
            @pl.loop(0, nblk)
            def _(blk):
                pltpu.sync_copy(
                    edges_hbm.at[pl.ds(base + blk * NBLK, NBLK)], idx_v)
                for b in range(NBUF):
                    pltpu.async_copy(y_hbm.at[idx_v.at[b, 0]], rows[b], gsem[b])

                @pl.loop(0, NBLK, step=NBUF)
                def _(m):
                    for b in range(NBUF):
                        pltpu.make_async_copy(y_hbm.at[idx_v.at[m + b, 0]],
                                              rows[b], gsem[b]).wait()
                        pltpu.async_copy(rows[b], z_sh.at[idx_v.at[m + b, 1]],
                                         ssem[b], add=True)
                    for b in range(NBUF):
                        pltpu.make_async_copy(rows[b],
                                              z_sh.at[idx_v.at[m + b, 1]],
                                              ssem[b]).wait()

                        @pl.when(m + NBUF + b < NBLK)
                        def _():
                            pltpu.async_copy(
                                y_hbm.at[idx_v.at[m + NBUF + b, 0]],
                                rows[b], gsem[b])

        @pl.when(cid == 0)
        def _():
            run(y0_hbm, Q0 // NBLK, sid * Q0)

        @pl.when(cid == 1)
        def _():
            run(y1_hbm, Q1 // NBLK, NS * Q0 + sid * Q1)

        plsc.subcore_barrier()

        pltpu.sync_copy(z_sh.at[pl.ds(sid * ROWS_PER_TILE, ROWS_PER_TILE)],
                        zp_hbm.at[cid, pl.ds(sid * ROWS_PER_TILE, ROWS_PER_TILE)])

    return k(edges2, y0, y1, zfull)


def _pass_e(x3, batp, zfull):
    @functools.partial(
        pl.kernel,
        out_type=jax.ShapeDtypeStruct((NC, NP, H), _f32),
        mesh=_mesh,
        scratch_types=[
            pltpu.VMEM((64,), _i32),
            pltpu.VMEM((64, H), _f32),
            pltpu.VMEM_SHARED((NP, H), _f32),
        ],
    )
    def k(x3_hbm, batp_hbm, zf_hbm, sp_hbm, bidx_v, rows_v, s_sh):
        cid = lax.axis_index("c")
        sid = lax.axis_index("s")
        wid = _wid()

        @pl.loop(0, ROWS_PER_TILE // 64)
        def _(m):
            off = sid * ROWS_PER_TILE + m * 64
            pltpu.sync_copy(zf_hbm.at[pl.ds(off, 64)], s_sh.at[pl.ds(off, 64)])

        plsc.subcore_barrier()

        @pl.loop(0, NP // NW // 64)
        def _(j):
            off = wid * (NP // NW) + j * 64
            pltpu.sync_copy(x3_hbm.at[pl.ds(off, 64)], rows_v)
            pltpu.sync_copy(batp_hbm.at[pl.ds(off, 64)], bidx_v)
            pltpu.sync_copy(rows_v, s_sh.at[bidx_v], add=True)

        plsc.subcore_barrier()

        pltpu.sync_copy(s_sh.at[pl.ds(sid * ROWS_PER_TILE, ROWS_PER_TILE)],
                        sp_hbm.at[cid, pl.ds(sid * ROWS_PER_TILE, ROWS_PER_TILE)])

    return k(x3, batp, zfull)


_BLK = 2048
_GRID = NP // _BLK

_row_spec = pl.BlockSpec((_BLK, H), lambda i: (i, 0))
_col_spec = pl.BlockSpec((_BLK, 1), lambda i: (i, 0))
_w_spec = pl.BlockSpec((H, H), lambda i: (0, 0))
_b_spec = pl.BlockSpec((1, H), lambda i: (0, 0))


def _b_body(x_ref, d0_ref, d1_ref, c0_ref, c1_ref,
            y_ref, y2_ref, dinv_ref, cntinv_ref):
    deg = d0_ref[...] + d1_ref[...] + 1.0
    dinv = lax.rsqrt(deg)
    dinv_ref[...] = dinv
    y = x_ref[...] * dinv
    y_ref[...] = y
    y2_ref[...] = y
    cnt = c0_ref[...] + c1_ref[...]
    cntinv_ref[...] = 1.0 / jnp.maximum(cnt, 1.0)


def _pass_b(x, d0, d1, c0, c1):
    return pl.pallas_call(
        _b_body,
        grid=(_GRID,),
        in_specs=[_row_spec, _col_spec, _col_spec, _col_spec, _col_spec],
        out_specs=(_row_spec, _row_spec, _col_spec, _col_spec),
        out_shape=(
            jax.ShapeDtypeStruct((NP, H), _f32),
            jax.ShapeDtypeStruct((NP, H), _f32),
            jax.ShapeDtypeStruct((NP, 1), _f32),
            jax.ShapeDtypeStruct((NP, 1), _f32),
        ),
    )(x, d0, d1, c0, c1)


def _d_body(z0_ref, z1_ref, y_ref, dinv_ref, w_ref, b_ref, *out_refs,
            relu, scale_out):
    dinv = dinv_ref[...]
    t = (z0_ref[...] + z1_ref[...] + y_ref[...]) * dinv
    m = jnp.dot(t, w_ref[...], preferred_element_type=_f32) + b_ref[...]
    if relu:
        m = jnp.maximum(m, 0.0)
    if scale_out:
        m = m * dinv
    for o_ref in out_refs:
        o_ref[...] = m


def _pass_d(z0, z1, y, dinv, w, b, relu, scale_out):
    n_out = 2 if scale_out else 1
    return pl.pallas_call(
        functools.partial(_d_body, relu=relu, scale_out=scale_out),
        grid=(_GRID,),
        in_specs=[_row_spec, _row_spec, _row_spec, _col_spec, _w_spec, _b_spec],
        out_specs=(_row_spec,) * n_out,
        out_shape=(jax.ShapeDtypeStruct((NP, H), _f32),) * n_out,
    )(z0, z1, y, dinv, w, b)


def _f_body(s0_ref, s1_ref, cntinv_ref, w_ref, b_ref, o_ref):
    t = (s0_ref[...] + s1_ref[...]) * cntinv_ref[...]
    o_ref[...] = jnp.dot(t, w_ref[...], preferred_element_type=_f32) + b_ref[...]


def _pass_f(s0, s1, cntinv, wl, bl):
    return pl.pallas_call(
        _f_body,
        grid=(_GRID,),
        in_specs=[_row_spec, _row_spec, _col_spec, _w_spec, _b_spec],
        out_specs=_row_spec,
        out_shape=jax.ShapeDtypeStruct((NP, H), _f32),
    )(s0, s1, cntinv, wl, bl)


def kernel(user_ids, item_ids, edge_index, batch, user_table, item_table,
           W1, b1, W2, b2, W3, b3, Wl, bl):
    srcp = jnp.concatenate([edge_index[0], jnp.zeros((EP - E,), _i32)])
    dstp = jnp.concatenate([edge_index[1], jnp.full((EP - E,), TRASH, _i32)])
    edges2 = jnp.stack([srcp.reshape(EP // 128, 128),
                        dstp.reshape(EP // 128, 128)], axis=1)
    batp = jnp.concatenate([batch, jnp.full((BP - N,), TRASH, _i32)])
    ones128 = jnp.ones((128,), _f32)
    zeros1d = jnp.zeros((ROWS_PER_TILE,), _f32)
    zfull = jnp.zeros((NP, H), _f32)

    x, degc, cntc = _pass_a(user_ids, item_ids, user_table, item_table,
                            dstp, batp, ones128, zeros1d)
    d0 = degc[0].reshape(NP, 1)
    d1 = degc[1].reshape(NP, 1)
    c0 = cntc[0].reshape(NP, 1)
    c1 = cntc[1].reshape(NP, 1)

    ya, yb, dinv, cntinv = _pass_b(x, d0, d1, c0, c1)

    for w, b, relu, scale_out in ((W1, b1, True, True),
                                  (W2, b2, True, True),
                                  (W3, b3, False, False)):
        zp = _pass_c(edges2, ya, yb, zfull)
        res = _pass_d(zp[0], zp[1], ya, dinv, w, b.reshape(1, H),
                      relu, scale_out)
        if scale_out:
            ya, yb = res
        else:
            x3 = res[0]

    sp = _pass_e(x3, batp, zfull)
    out = _pass_f(sp[0], sp[1], cntinv, Wl, bl.reshape(1, H))
    return (out[:NU], out[NU:N])


# speedup vs baseline: 1.4658x; 1.0378x over previous
import functools

import jax
import jax.numpy as jnp
from jax import lax
from jax.experimental import pallas as pl
from jax.experimental.pallas import tpu as pltpu
from jax.experimental.pallas import tpu_sc as plsc

NU = 5000
NI = 5000
N = 10000
H = 128
E = 320000
G = 10000

NC = 2
NS = 16
NW = NC * NS

NP = 10240
EP = 327680
EW = EP // NW
BP = 12288
TRASH = 10000

ROWS_PER_TILE = NP // NS

_mesh = plsc.VectorSubcoreMesh(core_axis_name="c", subcore_axis_name="s")

_f32 = jnp.float32
_i32 = jnp.int32


def _wid():
    return lax.axis_index("s") * NC + lax.axis_index("c")


def _pass_a(user_ids, item_ids, user_table, item_table, dstp, batp,
            ones128, zeros1d):
    @functools.partial(
        pl.kernel,
        out_type=(
            jax.ShapeDtypeStruct((NP, H), _f32),
            jax.ShapeDtypeStruct((NC, NP), _f32),
            jax.ShapeDtypeStruct((NC, NP), _f32),
        ),
        mesh=_mesh,
        scratch_types=[
            pltpu.VMEM((40,), _i32),
            pltpu.VMEM((40, H), _f32),
            pltpu.VMEM((128,), _i32),
            pltpu.VMEM((128,), _f32),
            pltpu.VMEM((ROWS_PER_TILE,), _f32),
            pltpu.VMEM_SHARED((NP,), _f32),
            pltpu.VMEM_SHARED((NP,), _f32),
        ],
    )
    def k(uid_hbm, iid_hbm, utab_hbm, itab_hbm, dstp_hbm, batp_hbm,
          ones_hbm, z1d_hbm, x_hbm, deg_hbm, cnt_hbm,
          idx_v, rows_v, eidx_v, ones_v, z1d_v, deg_sh, cnt_sh):
        cid = lax.axis_index("c")
        sid = lax.axis_index("s")
        wid = _wid()

        pltpu.sync_copy(z1d_hbm, z1d_v)
        pltpu.sync_copy(z1d_v, deg_sh.at[pl.ds(sid * ROWS_PER_TILE, ROWS_PER_TILE)])
        pltpu.sync_copy(z1d_v, cnt_sh.at[pl.ds(sid * ROWS_PER_TILE, ROWS_PER_TILE)])
        pltpu.sync_copy(ones_hbm, ones_v)

        @pl.when(wid < 16)
        def _():
            @pl.loop(0, 8)
            def _(j):
                c = wid * 8 + j

                @pl.when(c < 125)
                def _():
                    pltpu.sync_copy(uid_hbm.at[pl.ds(c * 40, 40)], idx_v)
                    pltpu.sync_copy(utab_hbm.at[idx_v], rows_v)
                    pltpu.sync_copy(rows_v, x_hbm.at[pl.ds(c * 40, 40)])

        @pl.when(wid >= 16)
        def _():
            @pl.loop(0, 8)
            def _(j):
                c = (wid - 16) * 8 + j

                @pl.when(c < 125)
                def _():
                    pltpu.sync_copy(iid_hbm.at[pl.ds(c * 40, 40)], idx_v)
                    pltpu.sync_copy(itab_hbm.at[idx_v], rows_v)
                    pltpu.sync_copy(rows_v, x_hbm.at[pl.ds(NU + c * 40, 40)])

        plsc.subcore_barrier()

        @pl.loop(0, EW // 128)
        def _(j):
            pltpu.sync_copy(dstp_hbm.at[pl.ds(wid * EW + j * 128, 128)], eidx_v)
            pltpu.sync_copy(ones_v, deg_sh.at[eidx_v], add=True)

        @pl.loop(0, BP // NW // 128)
        def _(j):
            pltpu.sync_copy(batp_hbm.at[pl.ds(wid * (BP // NW) + j * 128, 128)], eidx_v)
            pltpu.sync_copy(ones_v, cnt_sh.at[eidx_v], add=True)

        plsc.subcore_barrier()

        @pl.when(sid == 0)
        def _():
            pltpu.sync_copy(deg_sh, deg_hbm.at[cid])
            pltpu.sync_copy(cnt_sh, cnt_hbm.at[cid])

    return k(user_ids, item_ids, user_table, item_table, dstp, batp,
             ones128, zeros1d)


NCHUNK = EW // 128
NBLK = 20
NBUF = 2
Q0 = 140
Q1 = 20


def _pass_c(edges2, y0, y1, zfull):
    @functools.partial(
        pl.kernel,
        out_type=jax.ShapeDtypeStruct((NC, NP, H), _f32),
        mesh=_mesh,
        scratch_types=[
            pltpu.VMEM((NBLK, 2, 128), _i32),
            pltpu.VMEM_SHARED((NP, H), _f32),
        ]
        + [pltpu.VMEM((128, H), _f32)] * NBUF
        + [pltpu.SemaphoreType.DMA] * (2 * NBUF),
    )
    def k(edges_hbm, y0_hbm, y1_hbm, zf_hbm, zp_hbm, idx_v, z_sh, *bufs_sems):
        rows = bufs_sems[:NBUF]
        gsem = bufs_sems[NBUF:2 * NBUF]
        ssem = bufs_sems[2 * NBUF:]
        cid = lax.axis_index("c")
        sid = lax.axis_index("s")

        @pl.loop(0, ROWS_PER_TILE // 64)
        def _(m):
            off = sid * ROWS_PER_TILE + m * 64
            pltpu.sync_copy(zf_hbm.at[pl.ds(off, 64)], z_sh.at[pl.ds(off, 64)])

        plsc.subcore_barrier()

        def run(y_hbm, nblk, base):
            @pl.loop(0, nblk)
            def _(blk):
                pltpu.sync_copy(
                    edges_hbm.at[pl.ds(base + blk * NBLK, NBLK)], idx_v)
                for b in range(NBUF):
                    pltpu.async_copy(y_hbm.at[idx_v.at[b, 0]], rows[b], gsem[b])

                @pl.loop(0, NBLK, step=NBUF)
                def _(m):
                    for b in range(NBUF):
                        pltpu.make_async_copy(y_hbm.at[idx_v.at[m + b, 0]],
                                              rows[b], gsem[b]).wait()
                        pltpu.async_copy(rows[b], z_sh.at[idx_v.at[m + b, 1]],
                                         ssem[b], add=True)
                    for b in range(NBUF):
                        pltpu.make_async_copy(rows[b],
                                              z_sh.at[idx_v.at[m + b, 1]],
                                              ssem[b]).wait()

                        @pl.when(m + NBUF + b < NBLK)
                        def _():
                            pltpu.async_copy(
                                y_hbm.at[idx_v.at[m + NBUF + b, 0]],
                                rows[b], gsem[b])

        @pl.when(cid == 0)
        def _():
            run(y0_hbm, Q0 // NBLK, sid * Q0)

        @pl.when(cid == 1)
        def _():
            run(y1_hbm, Q1 // NBLK, NS * Q0 + sid * Q1)

        plsc.subcore_barrier()

        pltpu.sync_copy(z_sh.at[pl.ds(sid * ROWS_PER_TILE, ROWS_PER_TILE)],
                        zp_hbm.at[cid, pl.ds(sid * ROWS_PER_TILE, ROWS_PER_TILE)])

    return k(edges2, y0, y1, zfull)


def _pass_e(x3, batp, zfull):
    @functools.partial(
        pl.kernel,
        out_type=jax.ShapeDtypeStruct((NC, NP, H), _f32),
        mesh=_mesh,
        scratch_types=[
            pltpu.VMEM((64,), _i32),
            pltpu.VMEM((64, H), _f32),
            pltpu.VMEM_SHARED((NP, H), _f32),
        ],
    )
    def k(x3_hbm, batp_hbm, zf_hbm, sp_hbm, bidx_v, rows_v, s_sh):
        cid = lax.axis_index("c")
        sid = lax.axis_index("s")
        wid = _wid()

        @pl.loop(0, ROWS_PER_TILE // 64)
        def _(m):
            off = sid * ROWS_PER_TILE + m * 64
            pltpu.sync_copy(zf_hbm.at[pl.ds(off, 64)], s_sh.at[pl.ds(off, 64)])

        plsc.subcore_barrier()

        @pl.loop(0, NP // NW // 64)
        def _(j):
            off = wid * (NP // NW) + j * 64
            pltpu.sync_copy(x3_hbm.at[pl.ds(off, 64)], rows_v)
            pltpu.sync_copy(batp_hbm.at[pl.ds(off, 64)], bidx_v)
            pltpu.sync_copy(rows_v, s_sh.at[bidx_v], add=True)

        plsc.subcore_barrier()

        pltpu.sync_copy(s_sh.at[pl.ds(sid * ROWS_PER_TILE, ROWS_PER_TILE)],
                        sp_hbm.at[cid, pl.ds(sid * ROWS_PER_TILE, ROWS_PER_TILE)])

    return k(x3, batp, zfull)


_BLK = 2048
_GRID = NP // _BLK

_row_spec = pl.BlockSpec((_BLK, H), lambda i: (i, 0))
_col_spec = pl.BlockSpec((_BLK, 1), lambda i: (i, 0))
_w_spec = pl.BlockSpec((H, H), lambda i: (0, 0))
_b_spec = pl.BlockSpec((1, H), lambda i: (0, 0))


def _b_body(x_ref, d0_ref, d1_ref, c0_ref, c1_ref,
            y_ref, y2_ref, dinv_ref, cntinv_ref):
    deg = d0_ref[...] + d1_ref[...] + 1.0
    dinv = lax.rsqrt(deg)
    dinv_ref[...] = dinv
    y = x_ref[...] * dinv
    y_ref[...] = y
    y2_ref[...] = y
    cnt = c0_ref[...] + c1_ref[...]
    cntinv_ref[...] = 1.0 / jnp.maximum(cnt, 1.0)


def _pass_b(x, d0, d1, c0, c1):
    return pl.pallas_call(
        _b_body,
        grid=(_GRID,),
        in_specs=[_row_spec, _col_spec, _col_spec, _col_spec, _col_spec],
        out_specs=(_row_spec, _row_spec, _col_spec, _col_spec),
        out_shape=(
            jax.ShapeDtypeStruct((NP, H), _f32),
            jax.ShapeDtypeStruct((NP, H), _f32),
            jax.ShapeDtypeStruct((NP, 1), _f32),
            jax.ShapeDtypeStruct((NP, 1), _f32),
        ),
    )(x, d0, d1, c0, c1)


def _d_body(z0_ref, z1_ref, y_ref, dinv_ref, w_ref, b_ref, *out_refs,
            relu, scale_out):
    dinv = dinv_ref[...]
    t = (z0_ref[...] + z1_ref[...] + y_ref[...]) * dinv
    m = jnp.dot(t, w_ref[...], preferred_element_type=_f32) + b_ref[...]
    if relu:
        m = jnp.maximum(m, 0.0)
    if scale_out:
        m = m * dinv
    for o_ref in out_refs:
        o_ref[...] = m


def _pass_d(z0, z1, y, dinv, w, b, relu, scale_out):
    n_out = 2 if scale_out else 1
    return pl.pallas_call(
        functools.partial(_d_body, relu=relu, scale_out=scale_out),
        grid=(_GRID,),
        in_specs=[_row_spec, _row_spec, _row_spec, _col_spec, _w_spec, _b_spec],
        out_specs=(_row_spec,) * n_out,
        out_shape=(jax.ShapeDtypeStruct((NP, H), _f32),) * n_out,
    )(z0, z1, y, dinv, w, b)


def _f_body(s0_ref, s1_ref, cntinv_ref, w_ref, b_ref, o_ref):
    t = (s0_ref[...] + s1_ref[...]) * cntinv_ref[...]
    o_ref[...] = jnp.dot(t, w_ref[...], preferred_element_type=_f32) + b_ref[...]


def _pass_f(s0, s1, cntinv, wl, bl):
    return pl.pallas_call(
        _f_body,
        grid=(_GRID,),
        in_specs=[_row_spec, _row_spec, _col_spec, _w_spec, _b_spec],
        out_specs=_row_spec,
        out_shape=jax.ShapeDtypeStruct((NP, H), _f32),
    )(s0, s1, cntinv, wl, bl)


def kernel(user_ids, item_ids, edge_index, batch, user_table, item_table,
           W1, b1, W2, b2, W3, b3, Wl, bl):
    srcp = jnp.concatenate([edge_index[0], jnp.zeros((EP - E,), _i32)])
    dstp = jnp.concatenate([edge_index[1], jnp.full((EP - E,), TRASH, _i32)])
    edges2 = jnp.stack([srcp.reshape(EP // 128, 128),
                        dstp.reshape(EP // 128, 128)], axis=1)
    batp = jnp.concatenate([batch, jnp.full((BP - N,), TRASH, _i32)])
    ones128 = jnp.ones((128,), _f32)
    zeros1d = jnp.zeros((ROWS_PER_TILE,), _f32)
    zfull = jnp.zeros((NP, H), _f32)

    x, degc, cntc = _pass_a(user_ids, item_ids, user_table, item_table,
                            dstp, batp, ones128, zeros1d)
    d0 = degc[0].reshape(NP, 1)
    d1 = degc[1].reshape(NP, 1)
    c0 = cntc[0].reshape(NP, 1)
    c1 = cntc[1].reshape(NP, 1)

    ya, yb, dinv, cntinv = _pass_b(x, d0, d1, c0, c1)

    for w, b, relu, scale_out in ((W1, b1, True, True),
                                  (W2, b2, True, True),
                                  (W3, b3, False, False)):
        zp = _pass_c(edges2, ya, yb, zfull)
        res = _pass_d(zp[0], zp[1], ya, dinv, w, b.reshape(1, H),
                      relu, scale_out)
        if scale_out:
            ya, yb = res
        else:
            x3 = res[0]

    sp = _pass_e(x3, batp, zfull)
    out = _pass_f(sp[0], sp[1], cntinv, Wl, bl.reshape(1, H))
    return (out[:NU], out[NU:N])
